# baseline (speedup 1.0000x reference)
import functools

import jax
import jax.numpy as jnp
from jax import lax
from jax.experimental import pallas as pl
from jax.experimental.pallas import tpu as pltpu
from jax.experimental.pallas import tpu_sc as plsc

N = 10000
E = 320000
D = 128
C = 128

ROW_BLK = 1024
N_PAD = 10240

NC = 2
NS = 16
NW = NC * NS
EPW = E // NW
VPC = 8
NCHUNK = EPW // (16 * VPC)
REM_V = (EPW - NCHUNK * 16 * VPC) // 16


def _err_node_body(h_ref, t_ref, w_ref, b_ref, out_ref):
    y = jnp.dot(h_ref[...], w_ref[...], preferred_element_type=jnp.float32)
    d = y + b_ref[...] - t_ref[...]
    e = jnp.mean(d * d, axis=1)
    out_ref[...] = e.reshape(ROW_BLK // 128, 128)


def _err_node(H, target, W, b):
    grid = (N_PAD // ROW_BLK,)
    return pl.pallas_call(
        _err_node_body,
        grid=grid,
        in_specs=[
            pl.BlockSpec((ROW_BLK, D), lambda g: (g, 0)),
            pl.BlockSpec((ROW_BLK, D), lambda g: (g, 0)),
            pl.BlockSpec((D, D), lambda g: (0, 0)),
            pl.BlockSpec((1, D), lambda g: (0, 0)),
        ],
        out_specs=pl.BlockSpec((ROW_BLK // 128, 128), lambda g: (g, 0)),
        out_shape=jax.ShapeDtypeStruct((N_PAD // 128, 128), jnp.float32),
    )(H, target, W, b.reshape(1, D))


def _sc_scatter_body(src_hbm, dst_hbm, err_hbm, centers_hbm, out_hbm,
                     src_v, dst_v, err_v, sum_v, cnt_v, cen_v, obuf_v, sem_a):
    cid = lax.axis_index("c")
    sid = lax.axis_index("s")
    wid = sid * NC + cid

    in_src = pltpu.async_copy(src_hbm.at[wid], src_v, sem_a)
    in_dst = pltpu.async_copy(dst_hbm.at[wid], dst_v, sem_a)
    in_err = pltpu.async_copy(err_hbm, err_v, sem_a)
    in_cen = pltpu.async_copy(centers_hbm, cen_v, sem_a)

    zero16 = jnp.zeros((16,), jnp.float32)

    def zero_body(i, carry):
        sum_v[pl.ds(i * 16, 16)] = zero16
        cnt_v[pl.ds(i * 16, 16)] = zero16
        return carry

    lax.fori_loop(0, N // 16, zero_body, 0)
    in_src.wait()
    in_dst.wait()
    in_err.wait()
    in_cen.wait()

    def one_vector(base):
        s16 = src_v[pl.ds(base, 16)]
        d16 = dst_v[pl.ds(base, 16)]
        e16 = plsc.load_gather(err_v, [d16])
        cnts, last = plsc.scan_count(s16)
        plsc.addupdate_scatter(cnt_v, [s16], cnts.astype(jnp.float32),
                               mask=last)
        plsc.addupdate_scatter(sum_v, [s16], e16, mask=last)
        return jnp.logical_not(last)

    def extra_rounds(base, leftover):
        s16 = src_v[pl.ds(base, 16)]
        d16 = dst_v[pl.ds(base, 16)]
        e16 = plsc.load_gather(err_v, [d16])

        def w_body(act):
            _, last2 = plsc.scan_count(s16, act)
            plsc.addupdate_scatter(sum_v, [s16], e16, mask=last2)
            return jnp.logical_and(act, jnp.logical_not(last2))

        lax.while_loop(lambda a: jnp.any(a), w_body, leftover)

    def chunk_body(j, nv):
        base = j * (16 * VPC)
        leftover = jnp.zeros((16,), jnp.bool_)
        for k in range(VPC):
            leftover = jnp.logical_or(leftover, one_vector(base + k * 16))

        @pl.when(jnp.any(leftover))
        def _():
            for k in range(VPC):
                extra_rounds(base + k * 16,
                             one_leftover(base + k * 16))
        return nv

    def one_leftover(base):
        s16 = src_v[pl.ds(base, 16)]
        _, last = plsc.scan_count(s16)
        return jnp.logical_not(last)

    lax.fori_loop(0, NCHUNK, chunk_body, 0)

    tail_base = NCHUNK * 16 * VPC
    leftover = jnp.zeros((16,), jnp.bool_)
    for k in range(REM_V):
        leftover = jnp.logical_or(leftover, one_vector(tail_base + k * 16))

    @pl.when(jnp.any(leftover))
    def _():
        for k in range(REM_V):
            extra_rounds(tail_base + k * 16, one_leftover(tail_base + k * 16))

    for k in range(C // 16):
        c16 = cen_v[pl.ds(k * 16, 16)]
        obuf_v[0, pl.ds(k * 16, 16)] = plsc.load_gather(sum_v, [c16])
        obuf_v[1, pl.ds(k * 16, 16)] = plsc.load_gather(cnt_v, [c16])
    pltpu.sync_copy(obuf_v, out_hbm.at[wid])


_sc_scatter = functools.partial(
    pl.kernel,
    _sc_scatter_body,
    out_type=jax.ShapeDtypeStruct((NW, 2, C), jnp.float32),
    mesh=plsc.VectorSubcoreMesh(core_axis_name="c", subcore_axis_name="s"),
    compiler_params=pltpu.CompilerParams(needs_layout_passes=False),
    scratch_types=[
        pltpu.VMEM((EPW,), jnp.int32),
        pltpu.VMEM((EPW,), jnp.int32),
        pltpu.VMEM((N_PAD,), jnp.float32),
        pltpu.VMEM((N,), jnp.float32),
        pltpu.VMEM((N,), jnp.float32),
        pltpu.VMEM((C,), jnp.int32),
        pltpu.VMEM((2, C), jnp.float32),
        pltpu.SemaphoreType.DMA,
    ],
)


def _final_body(p_ref, out_ref):
    p = p_ref[...]
    loss_sum = jnp.sum(p[:, 0, :], axis=0, keepdims=True)
    cnt = jnp.sum(p[:, 1, :], axis=0, keepdims=True)
    aux = jnp.sum(loss_sum / jnp.maximum(cnt, 1.0)) * (1.0 / C)
    pairs = jnp.sum(cnt)
    mpl = jnp.sum(loss_sum) / pairs
    mdeg = jnp.max(cnt)
    lane = lax.broadcasted_iota(jnp.int32, (1, C), 1)
    row = jnp.where(lane == 0, aux,
                    jnp.where(lane == 1, pairs,
                              jnp.where(lane == 2, mpl, mdeg)))
    out_ref[...] = row


def _finalize(partials):
    return pl.pallas_call(
        _final_body,
        out_shape=jax.ShapeDtypeStruct((1, C), jnp.float32),
    )(partials)


def kernel(H, edge_index, centers, target, W, b):
    H = H.astype(jnp.float32)
    target = target.astype(jnp.float32)
    W = W.astype(jnp.float32)
    b = b.astype(jnp.float32)
    edges = edge_index.astype(jnp.int32)
    centers = centers.astype(jnp.int32)

    err2d = _err_node(H, target, W, b)
    err_flat = err2d.reshape(N_PAD)

    src2 = edges[0].reshape(NW, EPW)
    dst2 = edges[1].reshape(NW, EPW)

    partials = _sc_scatter()(src2, dst2, err_flat, centers)
    row = _finalize(partials)

    aux_loss = row[0, 0]
    stats_pairs = row[0, 1]
    mean_pair_loss = row[0, 2]
    max_deg = row[0, 3]
    stats_centers = jnp.asarray(float(C), dtype=jnp.float32)
    return (aux_loss, stats_centers, stats_pairs, mean_pair_loss, max_deg)
